# Initial kernel scaffold; baseline (speedup 1.0000x reference)
import jax
import jax.numpy as jnp
from jax.experimental import pallas as pl


def kernel(x, router_w, router_bias, gate_w, up_w, down_w, shared_gate_w, shared_up_w, shared_down_w):
    raise NotImplementedError("write your pallas kernel here")


# speedup vs baseline: 1.8035x; 1.8035x over previous
import functools

import jax
import jax.numpy as jnp
from jax.experimental import pallas as pl
from jax.experimental.pallas import tpu as pltpu


def _moe_body(x_ref, rw_ref, rb_ref, gate_ref, up_ref, down_ref,
              sg_ref, su_ref, sd_ref, out_ref):
    E = rw_ref.shape[0]
    xb32 = x_ref[...]
    xb = xb32.astype(jnp.bfloat16)

    logits = jax.lax.dot_general(
        xb, rw_ref[...], (((1,), (1,)), ((), ())),
        preferred_element_type=jnp.float32)
    lb = logits + rb_ref[...]
    ex = jnp.exp(logits - jnp.max(logits, axis=-1, keepdims=True))
    scores = ex / jnp.sum(ex, axis=-1, keepdims=True)
    i1 = jnp.argmax(lb, axis=-1, keepdims=True)
    eiota = jax.lax.broadcasted_iota(jnp.int32, logits.shape, 1)
    masked = jnp.where(eiota == i1, -jnp.inf, lb)
    i2 = jnp.argmax(masked, axis=-1, keepdims=True)
    m1 = eiota == i1
    m2 = eiota == i2
    s1 = jnp.sum(jnp.where(m1, scores, 0.0), axis=-1, keepdims=True)
    s2 = jnp.sum(jnp.where(m2, scores, 0.0), axis=-1, keepdims=True)
    denom = s1 + s2
    combine = (jnp.where(m1, s1, 0.0) + jnp.where(m2, s2, 0.0)) / denom

    acc = jnp.zeros(out_ref.shape, jnp.float32)
    for e in range(E):
        g = jax.lax.dot_general(
            xb, gate_ref[e], (((1,), (1,)), ((), ())),
            preferred_element_type=jnp.float32)
        u = jax.lax.dot_general(
            xb, up_ref[e], (((1,), (1,)), ((), ())),
            preferred_element_type=jnp.float32)
        h = (g * jax.lax.logistic(g) * u).astype(jnp.bfloat16)
        eo = jax.lax.dot_general(
            h, down_ref[e], (((1,), (0,)), ((), ())),
            preferred_element_type=jnp.float32)
        acc = acc + combine[:, e:e + 1] * eo

    gs = jax.lax.dot_general(
        xb, sg_ref[...], (((1,), (1,)), ((), ())),
        preferred_element_type=jnp.float32)
    us = jax.lax.dot_general(
        xb, su_ref[...], (((1,), (1,)), ((), ())),
        preferred_element_type=jnp.float32)
    hs = (gs * jax.lax.logistic(gs) * us).astype(jnp.bfloat16)
    acc = acc + jax.lax.dot_general(
        hs, sd_ref[...], (((1,), (0,)), ((), ())),
        preferred_element_type=jnp.float32)

    out_ref[...] = acc


@functools.partial(jax.jit, static_argnames=())
def kernel(x, router_w, router_bias, gate_w, up_w, down_w,
           shared_gate_w, shared_up_w, shared_down_w):
    Bs, Ts, D = x.shape
    N = Bs * Ts
    E, H, _ = gate_w.shape
    NS, SH, _ = shared_gate_w.shape
    TB = 512 if N % 512 == 0 else N
    flat = x.reshape(N, D)

    bf = jnp.bfloat16
    rw = router_w.astype(bf)
    gw = gate_w.astype(bf)
    uw = up_w.astype(bf)
    dw = jnp.swapaxes(down_w, 1, 2).astype(bf)
    sg = shared_gate_w.reshape(NS * SH, D).astype(bf)
    su = shared_up_w.reshape(NS * SH, D).astype(bf)
    sd = jnp.swapaxes(shared_down_w, 1, 2).reshape(NS * SH, D).astype(bf)

    grid = (N // TB,)
    out = pl.pallas_call(
        _moe_body,
        grid=grid,
        in_specs=[
            pl.BlockSpec((TB, D), lambda i: (i, 0)),
            pl.BlockSpec((E, D), lambda i: (0, 0)),
            pl.BlockSpec((E,), lambda i: (0,)),
            pl.BlockSpec((E, H, D), lambda i: (0, 0, 0)),
            pl.BlockSpec((E, H, D), lambda i: (0, 0, 0)),
            pl.BlockSpec((E, H, D), lambda i: (0, 0, 0)),
            pl.BlockSpec((NS * SH, D), lambda i: (0, 0)),
            pl.BlockSpec((NS * SH, D), lambda i: (0, 0)),
            pl.BlockSpec((NS * SH, D), lambda i: (0, 0)),
        ],
        out_specs=pl.BlockSpec((TB, D), lambda i: (i, 0)),
        out_shape=jax.ShapeDtypeStruct((N, D), jnp.float32),
        compiler_params=pltpu.CompilerParams(
            dimension_semantics=("arbitrary",),
        ),
    )(flat, rw, router_bias, gw, uw, dw, sg, su, sd)
    return out.reshape(Bs, Ts, D)
